# Initial kernel scaffold; baseline (speedup 1.0000x reference)
import jax
import jax.numpy as jnp
from jax.experimental import pallas as pl


def kernel(lmf, umf):
    raise NotImplementedError("write your pallas kernel here")


# speedup vs baseline: 174.6904x; 174.6904x over previous
import jax
import jax.numpy as jnp
from jax import lax
from jax.experimental import pallas as pl
from jax.experimental.pallas import tpu as pltpu
from jax.experimental.pallas import tpu_sc as plsc

B = 64
S = 4
D = 8
R = S ** D
NC = 2
NS = 16
NW = NC * NS
PER_W = 2 * B // NW

_DNUMS = lax.GatherDimensionNumbers(
    offset_dims=(), collapsed_slice_dims=(0,), start_index_map=(0,))


def _take(v, idx):
    return lax.gather(v, idx[:, None], _DNUMS, (1,),
                      mode=lax.GatherScatterMode.PROMISE_IN_BOUNDS)


def _sc_body(x_hbm, out_hbm, in_v, row_v):
    cid = lax.axis_index("c")
    sid = lax.axis_index("s")
    wid = sid * NC + cid

    iota = lax.iota(jnp.int32, 16)
    hi = iota >> 2
    lo = iota & 3
    zeros = jnp.zeros((16,), jnp.int32)

    for k in range(PER_W):
        task = wid + NW * k
        t = task // B
        b = task % B

        pltpu.sync_copy(x_hbm.at[t, b], in_v)
        vlow = in_v[pl.ds(0, 16)]
        vhigh = in_v[pl.ds(16, 16)]

        t01 = _take(vlow, hi) * _take(vlow, 4 + lo)
        t23 = _take(vlow, 8 + hi) * _take(vlow, 12 + lo)
        t45 = _take(vhigh, hi) * _take(vhigh, 4 + lo)
        t67 = _take(vhigh, 8 + hi) * _take(vhigh, 12 + lo)

        c_regs = [_take(t45, jnp.full((16,), j, jnp.int32)) * t67
                  for j in range(16)]

        def row_body(i, carry):
            a_splat = (_take(t01, zeros + (i >> 4)) *
                       _take(t23, zeros + (i & 15)))
            base = i * 256
            for j in range(16):
                row_v[pl.ds(base + 16 * j, 16)] = a_splat * c_regs[j]
            return carry

        lax.fori_loop(0, 256, row_body, 0)

        pltpu.sync_copy(row_v, out_hbm.at[t, b])


@jax.jit
def kernel(lmf, umf):
    x = jnp.stack([lmf, umf]).transpose(0, 1, 3, 2).reshape(2, B, S * D)
    mesh = plsc.VectorSubcoreMesh(core_axis_name="c", subcore_axis_name="s")
    out = pl.kernel(
        _sc_body,
        out_type=jax.ShapeDtypeStruct((2, B, R), jnp.float32),
        mesh=mesh,
        scratch_types=[
            pltpu.VMEM((S * D,), jnp.float32),
            pltpu.VMEM((R,), jnp.float32),
        ],
    )(x)
    return out[0], out[1]


# speedup vs baseline: 260.9778x; 1.4939x over previous
import jax
import jax.numpy as jnp
from jax import lax
from jax.experimental import pallas as pl
from jax.experimental.pallas import tpu as pltpu
from jax.experimental.pallas import tpu_sc as plsc

B = 64
S = 4
D = 8
R = S ** D
NC = 2
NS = 16
NW = NC * NS
PER_W = 2 * B // NW

_DNUMS = lax.GatherDimensionNumbers(
    offset_dims=(), collapsed_slice_dims=(0,), start_index_map=(0,))


def _take(v, idx):
    return lax.gather(v, idx[:, None], _DNUMS, (1,),
                      mode=lax.GatherScatterMode.PROMISE_IN_BOUNDS)


def _sc_body(l_hbm, u_hbm, out_l, out_u, in_v, row_v):
    cid = lax.axis_index("c")
    sid = lax.axis_index("s")
    wid = sid * NC + cid

    iota = lax.iota(jnp.int32, 16)
    hi = iota >> 2
    lo = iota & 3
    zeros = jnp.zeros((16,), jnp.int32)

    for x_hbm, o_hbm in ((l_hbm, out_l), (u_hbm, out_u)):
      for k in range(B // NW):
        b = wid + NW * k

        pltpu.sync_copy(x_hbm.at[b], in_v)
        vlow = in_v[pl.ds(0, 16)]
        vhigh = in_v[pl.ds(16, 16)]


        t01 = _take(vlow, hi) * _take(vlow, 4 + lo)
        t23 = _take(vlow, 8 + hi) * _take(vlow, 12 + lo)
        t45 = _take(vhigh, hi) * _take(vhigh, 4 + lo)
        t67 = _take(vhigh, 8 + hi) * _take(vhigh, 12 + lo)

        c_regs = [_take(t45, jnp.full((16,), j, jnp.int32)) * t67
                  for j in range(16)]

        def row_body(i, carry):
            a_splat = (_take(t01, zeros + (i >> 4)) *
                       _take(t23, zeros + (i & 15)))
            base = i * 256
            for j in range(16):
                row_v[pl.ds(base + 16 * j, 16)] = a_splat * c_regs[j]
            return carry

        lax.fori_loop(0, 256, row_body, 0)

        pltpu.sync_copy(row_v, o_hbm.at[b])


@jax.jit
def kernel(lmf, umf):
    lt = lmf.transpose(0, 2, 1).reshape(B, S * D)
    ut = umf.transpose(0, 2, 1).reshape(B, S * D)
    mesh = plsc.VectorSubcoreMesh(core_axis_name="c", subcore_axis_name="s")
    out_l, out_u = pl.kernel(
        _sc_body,
        out_type=(jax.ShapeDtypeStruct((B, R), jnp.float32),
                  jax.ShapeDtypeStruct((B, R), jnp.float32)),
        mesh=mesh,
        scratch_types=[
            pltpu.VMEM((S * D,), jnp.float32),
            pltpu.VMEM((R,), jnp.float32),
        ],
    )(lt, ut)
    return out_l, out_u


# speedup vs baseline: 294.9002x; 1.1300x over previous
import jax
import jax.numpy as jnp
from jax import lax
from jax.experimental import pallas as pl
from jax.experimental.pallas import tpu as pltpu
from jax.experimental.pallas import tpu_sc as plsc

B = 64
S = 4
D = 8
R = S ** D
NC = 2
NS = 16
NW = NC * NS
PER_W = 2 * B // NW

_DNUMS = lax.GatherDimensionNumbers(
    offset_dims=(), collapsed_slice_dims=(0,), start_index_map=(0,))


def _take(v, idx):
    return lax.gather(v, idx[:, None], _DNUMS, (1,),
                      mode=lax.GatherScatterMode.PROMISE_IN_BOUNDS)


CH = 8192
NCHUNK = R // CH


def _sc_body(l_hbm, u_hbm, out_l, out_u, in_v, buf0, buf1, sem0, sem1):
    cid = lax.axis_index("c")
    sid = lax.axis_index("s")
    wid = sid * NC + cid

    iota = lax.iota(jnp.int32, 16)
    hi = iota >> 2
    lo = iota & 3
    zeros = jnp.zeros((16,), jnp.int32)

    bufs = (buf0, buf1)
    sems = (sem0, sem1)
    pending = [None, None]
    gch = 0

    for x_hbm, o_hbm in ((l_hbm, out_l), (u_hbm, out_u)):
      for k in range(B // NW):
        b = wid + NW * k

        pltpu.sync_copy(x_hbm.at[b], in_v)
        vlow = in_v[pl.ds(0, 16)]
        vhigh = in_v[pl.ds(16, 16)]


        t01 = _take(vlow, hi) * _take(vlow, 4 + lo)
        t23 = _take(vlow, 8 + hi) * _take(vlow, 12 + lo)
        t45 = _take(vhigh, hi) * _take(vhigh, 4 + lo)
        t67 = _take(vhigh, 8 + hi) * _take(vhigh, 12 + lo)

        c_regs = [_take(t45, jnp.full((16,), j, jnp.int32)) * t67
                  for j in range(16)]

        for ch in range(NCHUNK):
            p = gch & 1
            gch += 1
            buf = bufs[p]
            if pending[p] is not None:
                pending[p].wait()

            def chunk_body(i, carry, _ch=ch, _buf=buf):
                ig = _ch * (CH // 256) + i
                a_splat = (_take(t01, zeros + (ig >> 4)) *
                           _take(t23, zeros + (ig & 15)))
                base = i * 256
                for j in range(16):
                    _buf[pl.ds(base + 16 * j, 16)] = a_splat * c_regs[j]
                return carry

            lax.fori_loop(0, CH // 256, chunk_body, 0)
            pending[p] = pltpu.async_copy(
                buf, o_hbm.at[b, pl.ds(ch * CH, CH)], sems[p])

    for p in range(2):
        if pending[p] is not None:
            pending[p].wait()


@jax.jit
def kernel(lmf, umf):
    lt = lmf.transpose(0, 2, 1).reshape(B, S * D)
    ut = umf.transpose(0, 2, 1).reshape(B, S * D)
    mesh = plsc.VectorSubcoreMesh(core_axis_name="c", subcore_axis_name="s")
    out_l, out_u = pl.kernel(
        _sc_body,
        out_type=(jax.ShapeDtypeStruct((B, R), jnp.float32),
                  jax.ShapeDtypeStruct((B, R), jnp.float32)),
        mesh=mesh,
        scratch_types=[
            pltpu.VMEM((S * D,), jnp.float32),
            pltpu.VMEM((CH,), jnp.float32),
            pltpu.VMEM((CH,), jnp.float32),
            pltpu.SemaphoreType.DMA,
            pltpu.SemaphoreType.DMA,
        ],
    )(lt, ut)
    return out_l, out_u
